# baseline (speedup 1.0000x reference)
import functools

import jax
import jax.numpy as jnp
from jax.experimental import pallas as pl
from jax.experimental.pallas import tpu as pltpu
from jax.experimental.pallas import tpu_sc as plsc

LANES = 128


def _route_body(x_ref, wgp_ref, bgp_ref, g_ref, posi_ref, wa_ref, wb_ref,
                meta_ref, cnt_sc, run_sc, off_sc, *, n_exp, tmb):
    p = pl.program_id(0)
    t = pl.program_id(1)
    tm = x_ref.shape[0]
    lanes = jax.lax.broadcasted_iota(jnp.int32, (tm, LANES), 1)

    logits = jnp.dot(x_ref[...], wgp_ref[...],
                     preferred_element_type=jnp.float32) + bgp_ref[...]
    mx = jnp.max(logits, axis=1, keepdims=True)
    ex = jnp.exp(logits - mx)
    gates = ex / jnp.sum(ex, axis=1, keepdims=True)
    m1 = jnp.max(gates, axis=1, keepdims=True)
    i1 = jnp.min(jnp.where(gates == m1, lanes, LANES), axis=1, keepdims=True)
    oh1 = lanes == i1
    rest = jnp.where(oh1, -1.0, gates)
    m2v = jnp.max(rest, axis=1, keepdims=True)
    i2 = jnp.min(jnp.where(rest == m2v, lanes, LANES), axis=1, keepdims=True)
    oh2 = lanes == i2
    oh1f = oh1.astype(jnp.float32)
    oh2f = oh2.astype(jnp.float32)
    ohsum = oh1f + oh2f
    tile_cnt = jnp.sum(ohsum, axis=0, keepdims=True)

    cr = jax.lax.broadcasted_iota(jnp.int32, (LANES, LANES), 0)
    cc = jax.lax.broadcasted_iota(jnp.int32, (LANES, LANES), 1)
    upper = (cr < cc).astype(jnp.float32)

    @pl.when((p == 0) & (t == 0))
    def _init():
        cnt_sc[...] = tile_cnt
        meta_ref[...] = jnp.zeros_like(meta_ref)

    @pl.when((p == 0) & (t > 0))
    def _count():
        cnt_sc[...] += tile_cnt

    @pl.when((p == 1) & (t == 0))
    def _offsets():
        cntp = jnp.ceil(cnt_sc[...] / tmb) * tmb
        off_sc[...] = jnp.dot(cntp, upper,
                              preferred_element_type=jnp.float32,
                              precision=jax.lax.Precision.HIGHEST)
        run_sc[...] = off_sc[...]

    @pl.when(p == 1)
    def _positions():
        s = m1 + m2v + 1e-12
        gm = jnp.where(oh1, m1 / s, 0.0) + jnp.where(oh2, m2v / s, 0.0)
        rr = jax.lax.broadcasted_iota(jnp.int32, (tm, tm), 0)
        rc = jax.lax.broadcasted_iota(jnp.int32, (tm, tm), 1)
        lstrict = (rc < rr).astype(jnp.float32)
        csum = jnp.dot(lstrict, ohsum, preferred_element_type=jnp.float32,
                       precision=jax.lax.Precision.HIGHEST)
        base = run_sc[...] + csum
        pos0 = jnp.sum(base * oh1f, axis=1, keepdims=True).astype(jnp.int32)
        pos1 = jnp.sum(base * oh2f, axis=1, keepdims=True).astype(jnp.int32)
        g_ref[...] = gm
        posi_ref[...] = jnp.where(lanes == 0, pos0,
                                  jnp.where(lanes == 1, pos1, 0))
        wa_ref[...] = jnp.broadcast_to(m1 / s, wa_ref.shape)
        wb_ref[...] = jnp.broadcast_to(m2v / s, wb_ref.shape)
        run_sc[...] += tile_cnt

    @pl.when((p == 1) & (t == pl.num_programs(1) - 1))
    def _meta():
        off = off_sc[...]
        cntp = jnp.ceil(cnt_sc[...] / tmb) * tmb
        ntile_row = cntp / tmb
        tstart_row = off / tmb
        tnext_row = tstart_row + ntile_row
        row8 = jax.lax.broadcasted_iota(jnp.int32, (8, LANES), 0)
        lane8 = jax.lax.broadcasted_iota(jnp.int32, (8, LANES), 1)
        diag = (lane8 == row8).astype(jnp.float32)
        tstart_col = jnp.sum(jnp.broadcast_to(tstart_row, (8, LANES)) * diag,
                             axis=1, keepdims=True)
        tnext_col = jnp.sum(jnp.broadcast_to(tnext_row, (8, LANES)) * diag,
                            axis=1, keepdims=True)
        lanef8 = lane8.astype(jnp.float32)
        belongs = ((lanef8 >= tstart_col) & (lanef8 < tnext_col)
                   & (row8 < n_exp))
        etask = jnp.sum(jnp.where(belongs, row8.astype(jnp.float32), 0.0),
                        axis=0, keepdims=True)
        tot = jnp.sum(ntile_row, axis=1, keepdims=True)
        lane1 = jax.lax.broadcasted_iota(jnp.int32, (1, LANES), 1)
        lanef1 = lane1.astype(jnp.float32)
        elast = jnp.max(jnp.where(cnt_sc[...] > 0, lanef1, 0.0),
                        axis=1, keepdims=True)
        validr = lanef1 < tot
        etaskf = jnp.where(validr, etask, elast)
        metam = jnp.where(
            row8 == 0, jnp.broadcast_to(etaskf, (8, LANES)),
            jnp.where(row8 == 1,
                      jnp.broadcast_to(validr.astype(jnp.float32), (8, LANES)),
                      0.0))
        meta_ref[...] = metam.astype(jnp.int32)


def _route(x, wgp, bgp, n_exp, tmb):
    b, d = x.shape
    tma = min(1024, b)
    body = functools.partial(_route_body, n_exp=n_exp, tmb=tmb)
    return pl.pallas_call(
        body,
        grid=(2, b // tma),
        in_specs=[
            pl.BlockSpec((tma, d), lambda p, t: (t, 0)),
            pl.BlockSpec((d, LANES), lambda p, t: (0, 0)),
            pl.BlockSpec((1, LANES), lambda p, t: (0, 0)),
        ],
        out_specs=[
            pl.BlockSpec((tma, LANES), lambda p, t: (t, 0)),
            pl.BlockSpec((tma, LANES), lambda p, t: (t, 0)),
            pl.BlockSpec((tma, 16), lambda p, t: (t, 0)),
            pl.BlockSpec((tma, 16), lambda p, t: (t, 0)),
            pl.BlockSpec((8, LANES), lambda p, t: (0, 0)),
        ],
        out_shape=[
            jax.ShapeDtypeStruct((b, LANES), jnp.float32),
            jax.ShapeDtypeStruct((b, LANES), jnp.int32),
            jax.ShapeDtypeStruct((b, 16), jnp.float32),
            jax.ShapeDtypeStruct((b, 16), jnp.float32),
            jax.ShapeDtypeStruct((8, LANES), jnp.int32),
        ],
        scratch_shapes=[
            pltpu.VMEM((1, LANES), jnp.float32),
            pltpu.VMEM((1, LANES), jnp.float32),
            pltpu.VMEM((1, LANES), jnp.float32),
        ],
        compiler_params=pltpu.CompilerParams(
            dimension_semantics=("arbitrary", "arbitrary")),
    )(x, wgp, bgp)


def _sc_scatter(x3, pos0, pos1, nrows):
    b, sl, ll = x3.shape
    nw = 32
    chunk = b // nw
    sub = 32
    nsub = chunk // sub
    mesh = plsc.VectorSubcoreMesh(core_axis_name="c", subcore_axis_name="s")

    @functools.partial(
        pl.kernel, mesh=mesh,
        out_type=jax.ShapeDtypeStruct((nrows, sl, ll), jnp.float32),
        scratch_types=[
            pltpu.VMEM((sub, sl, ll), jnp.float32),
            pltpu.VMEM((sub,), jnp.int32),
            pltpu.VMEM((sub,), jnp.int32),
            pltpu.SemaphoreType.DMA,
        ],
    )
    def scat(x_hbm, p0_hbm, p1_hbm, xs_hbm, rows_v, i0_v, i1_v, sem):
        wid = jax.lax.axis_index("s") * 2 + jax.lax.axis_index("c")
        base = wid * chunk

        def body(si, _):
            off = base + si * sub
            pltpu.sync_copy(x_hbm.at[pl.ds(off, sub)], rows_v)
            pltpu.sync_copy(p0_hbm.at[pl.ds(off, sub)], i0_v)
            pltpu.sync_copy(p1_hbm.at[pl.ds(off, sub)], i1_v)
            cp0 = pltpu.async_copy(rows_v, xs_hbm.at[i0_v], sem)
            cp1 = pltpu.async_copy(rows_v, xs_hbm.at[i1_v], sem)
            cp0.wait()
            cp1.wait()
            return ()

        jax.lax.fori_loop(0, nsub, body, ())

    return scat(x3, pos0, pos1)


def _gmm_body(meta_ref, xs_ref, w1_ref, b1_ref, w2_ref, b2_ref, out_ref):
    t = pl.program_id(0)

    @pl.when(meta_ref[1, t] == 1)
    def _compute():
        nk = xs_ref.shape[1]
        acc = jnp.dot(xs_ref[:, 0, :].astype(jnp.bfloat16), w1_ref[0, 0],
                      preferred_element_type=jnp.float32)
        for k in range(1, nk):
            acc += jnp.dot(xs_ref[:, k, :].astype(jnp.bfloat16), w1_ref[0, k],
                           preferred_element_type=jnp.float32)
        h = jnp.maximum(acc + b1_ref[0], 0.0).astype(jnp.bfloat16)
        out_ref[...] = (jnp.dot(h, w2_ref[0],
                                preferred_element_type=jnp.float32)
                        + b2_ref[0])


def _gmm(meta, xs3, w1r, b1r, w2q, b2q, tmb):
    nrows, nk, ll = xs3.shape
    h = w1r.shape[3]
    ov = w2q.shape[2]
    nt = nrows // tmb
    return pl.pallas_call(
        _gmm_body,
        grid_spec=pltpu.PrefetchScalarGridSpec(
            num_scalar_prefetch=1,
            grid=(nt,),
            in_specs=[
                pl.BlockSpec((tmb, nk, ll), lambda t, meta: (t, 0, 0)),
                pl.BlockSpec((1, nk, ll, h),
                             lambda t, meta: (meta[0, t], 0, 0, 0)),
                pl.BlockSpec((1, 1, h), lambda t, meta: (meta[0, t], 0, 0)),
                pl.BlockSpec((1, h, ov), lambda t, meta: (meta[0, t], 0, 0)),
                pl.BlockSpec((1, 1, ov), lambda t, meta: (meta[0, t], 0, 0)),
            ],
            out_specs=pl.BlockSpec((tmb, ov), lambda t, meta: (t, 0)),
        ),
        out_shape=jax.ShapeDtypeStruct((nrows, ov), jnp.float32),
        compiler_params=pltpu.CompilerParams(
            dimension_semantics=("arbitrary",)),
    )(meta, xs3, w1r, b1r, w2q, b2q)


def _sc_combine(outg, pos0, pos1, wa, wb):
    bk, ov = outg.shape
    b = pos0.shape[0]
    nw = 32
    chunk = b // nw
    mesh = plsc.VectorSubcoreMesh(core_axis_name="c", subcore_axis_name="s")

    @functools.partial(
        pl.kernel, mesh=mesh,
        out_type=jax.ShapeDtypeStruct((b, 16), jnp.float32),
        scratch_types=[
            pltpu.VMEM((chunk, ov), jnp.float32),
            pltpu.VMEM((chunk, ov), jnp.float32),
            pltpu.VMEM((chunk, 16), jnp.float32),
            pltpu.VMEM((chunk,), jnp.int32),
            pltpu.VMEM((chunk,), jnp.int32),
            pltpu.VMEM((chunk, 16), jnp.float32),
            pltpu.VMEM((chunk, 16), jnp.float32),
            pltpu.SemaphoreType.DMA,
        ],
    )
    def comb(outg_hbm, p0_hbm, p1_hbm, wa_hbm, wb_hbm, out_hbm,
             r0_v, r1_v, o_v, i0_v, i1_v, wa_v, wb_v, sem):
        wid = jax.lax.axis_index("s") * 2 + jax.lax.axis_index("c")
        base = wid * chunk
        pltpu.sync_copy(p0_hbm.at[pl.ds(base, chunk)], i0_v)
        pltpu.sync_copy(p1_hbm.at[pl.ds(base, chunk)], i1_v)
        pltpu.sync_copy(wa_hbm.at[pl.ds(base, chunk)], wa_v)
        pltpu.sync_copy(wb_hbm.at[pl.ds(base, chunk)], wb_v)
        cp0 = pltpu.async_copy(outg_hbm.at[i0_v], r0_v, sem)
        cp1 = pltpu.async_copy(outg_hbm.at[i1_v], r1_v, sem)
        cp0.wait()
        cp1.wait()

        def body(i, _):
            o_v[i] = (wa_v[i] * r0_v[i, pl.ds(0, 16)]
                      + wb_v[i] * r1_v[i, pl.ds(0, 16)])
            return ()

        jax.lax.fori_loop(0, chunk, body, ())
        pltpu.sync_copy(o_v, out_hbm.at[pl.ds(base, chunk)])

    return comb(outg, pos0, pos1, wa, wb)


def kernel(x, Wg, bg, W1, b1, W2, b2):
    b, d = x.shape
    n_exp = Wg.shape[1]
    h = W1.shape[2]
    o = W2.shape[2]
    ov = LANES
    tmb = 256
    nrows = 2 * b + n_exp * tmb

    wgp = jnp.pad(Wg, ((0, 0), (0, LANES - n_exp)))
    bgp = jnp.pad(bg, (0, LANES - n_exp), constant_values=-1e30)[None, :]
    w2q = jnp.pad(W2, ((0, 0), (0, 0), (0, ov - o))).astype(jnp.bfloat16)
    b1r = b1[:, None, :]
    b2q = jnp.pad(b2, ((0, 0), (0, ov - o)))[:, None, :]

    gpad, posi, wa, wb, meta = _route(x, wgp, bgp, n_exp, tmb)
    pos0 = posi[:, 0]
    pos1 = posi[:, 1]

    x3 = x.reshape(b, d // LANES, LANES)
    w1r = W1.astype(jnp.bfloat16).reshape(n_exp, d // LANES, LANES, h)
    xs3 = _sc_scatter(x3, pos0, pos1, nrows)
    outg = _gmm(meta, xs3, w1r, b1r, w2q, b2q, tmb)
    outw = _sc_combine(outg, pos0, pos1, wa, wb)
    return outw[:, :o], gpad[:, :n_exp]


# speedup vs baseline: 1.0954x; 1.0954x over previous
import functools

import jax
import jax.numpy as jnp
from jax.experimental import pallas as pl
from jax.experimental.pallas import tpu as pltpu
from jax.experimental.pallas import tpu_sc as plsc

LANES = 128


def _route_body(x_ref, wgp_ref, bgp_ref, g_ref, posi_ref, wa_ref, wb_ref,
                meta_ref, cnt_sc, run_sc, off_sc, *, n_exp, tmb):
    p = pl.program_id(0)
    t = pl.program_id(1)
    tm = x_ref.shape[0]
    lanes = jax.lax.broadcasted_iota(jnp.int32, (tm, LANES), 1)

    logits = jnp.dot(x_ref[...], wgp_ref[...],
                     preferred_element_type=jnp.float32) + bgp_ref[...]
    mx = jnp.max(logits, axis=1, keepdims=True)
    ex = jnp.exp(logits - mx)
    gates = ex / jnp.sum(ex, axis=1, keepdims=True)
    m1 = jnp.max(gates, axis=1, keepdims=True)
    i1 = jnp.min(jnp.where(gates == m1, lanes, LANES), axis=1, keepdims=True)
    oh1 = lanes == i1
    rest = jnp.where(oh1, -1.0, gates)
    m2v = jnp.max(rest, axis=1, keepdims=True)
    i2 = jnp.min(jnp.where(rest == m2v, lanes, LANES), axis=1, keepdims=True)
    oh2 = lanes == i2
    oh1f = oh1.astype(jnp.float32)
    oh2f = oh2.astype(jnp.float32)
    ohsum = oh1f + oh2f
    tile_cnt = jnp.sum(ohsum, axis=0, keepdims=True)

    cr = jax.lax.broadcasted_iota(jnp.int32, (LANES, LANES), 0)
    cc = jax.lax.broadcasted_iota(jnp.int32, (LANES, LANES), 1)
    upper = (cr < cc).astype(jnp.float32)

    @pl.when((p == 0) & (t == 0))
    def _init():
        cnt_sc[...] = tile_cnt
        meta_ref[...] = jnp.zeros_like(meta_ref)

    @pl.when((p == 0) & (t > 0))
    def _count():
        cnt_sc[...] += tile_cnt

    @pl.when((p == 1) & (t == 0))
    def _offsets():
        cntp = jnp.ceil(cnt_sc[...] / tmb) * tmb
        off_sc[...] = jnp.dot(cntp, upper,
                              preferred_element_type=jnp.float32,
                              precision=jax.lax.Precision.HIGHEST)
        run_sc[...] = off_sc[...]

    @pl.when(p == 1)
    def _positions():
        s = m1 + m2v + 1e-12
        gm = jnp.where(oh1, m1 / s, 0.0) + jnp.where(oh2, m2v / s, 0.0)
        rr = jax.lax.broadcasted_iota(jnp.int32, (tm, tm), 0)
        rc = jax.lax.broadcasted_iota(jnp.int32, (tm, tm), 1)
        lstrict = (rc < rr).astype(jnp.float32)
        csum = jnp.dot(lstrict, ohsum, preferred_element_type=jnp.float32,
                       precision=jax.lax.Precision.HIGHEST)
        base = run_sc[...] + csum
        pos0 = jnp.sum(base * oh1f, axis=1, keepdims=True).astype(jnp.int32)
        pos1 = jnp.sum(base * oh2f, axis=1, keepdims=True).astype(jnp.int32)
        g_ref[...] = gm
        posi_ref[...] = jnp.where(lanes == 0, pos0,
                                  jnp.where(lanes == 1, pos1, 0))
        wa_ref[...] = jnp.broadcast_to(m1 / s, wa_ref.shape)
        wb_ref[...] = jnp.broadcast_to(m2v / s, wb_ref.shape)
        run_sc[...] += tile_cnt

    @pl.when((p == 1) & (t == pl.num_programs(1) - 1))
    def _meta():
        off = off_sc[...]
        cntp = jnp.ceil(cnt_sc[...] / tmb) * tmb
        ntile_row = cntp / tmb
        tstart_row = off / tmb
        tnext_row = tstart_row + ntile_row
        row8 = jax.lax.broadcasted_iota(jnp.int32, (8, LANES), 0)
        lane8 = jax.lax.broadcasted_iota(jnp.int32, (8, LANES), 1)
        diag = (lane8 == row8).astype(jnp.float32)
        tstart_col = jnp.sum(jnp.broadcast_to(tstart_row, (8, LANES)) * diag,
                             axis=1, keepdims=True)
        tnext_col = jnp.sum(jnp.broadcast_to(tnext_row, (8, LANES)) * diag,
                            axis=1, keepdims=True)
        lanef8 = lane8.astype(jnp.float32)
        belongs = ((lanef8 >= tstart_col) & (lanef8 < tnext_col)
                   & (row8 < n_exp))
        etask = jnp.sum(jnp.where(belongs, row8.astype(jnp.float32), 0.0),
                        axis=0, keepdims=True)
        tot = jnp.sum(ntile_row, axis=1, keepdims=True)
        lane1 = jax.lax.broadcasted_iota(jnp.int32, (1, LANES), 1)
        lanef1 = lane1.astype(jnp.float32)
        elast = jnp.max(jnp.where(cnt_sc[...] > 0, lanef1, 0.0),
                        axis=1, keepdims=True)
        validr = lanef1 < tot
        etaskf = jnp.where(validr, etask, elast)
        metam = jnp.where(
            row8 == 0, jnp.broadcast_to(etaskf, (8, LANES)),
            jnp.where(row8 == 1,
                      jnp.broadcast_to(validr.astype(jnp.float32), (8, LANES)),
                      0.0))
        meta_ref[...] = metam.astype(jnp.int32)


def _route(x, wgp, bgp, n_exp, tmb):
    b, d = x.shape
    tma = min(1024, b)
    body = functools.partial(_route_body, n_exp=n_exp, tmb=tmb)
    return pl.pallas_call(
        body,
        grid=(2, b // tma),
        in_specs=[
            pl.BlockSpec((tma, d), lambda p, t: (t, 0)),
            pl.BlockSpec((d, LANES), lambda p, t: (0, 0)),
            pl.BlockSpec((1, LANES), lambda p, t: (0, 0)),
        ],
        out_specs=[
            pl.BlockSpec((tma, LANES), lambda p, t: (t, 0)),
            pl.BlockSpec((tma, LANES), lambda p, t: (t, 0)),
            pl.BlockSpec((tma, 16), lambda p, t: (t, 0)),
            pl.BlockSpec((tma, 16), lambda p, t: (t, 0)),
            pl.BlockSpec((8, LANES), lambda p, t: (0, 0)),
        ],
        out_shape=[
            jax.ShapeDtypeStruct((b, LANES), jnp.float32),
            jax.ShapeDtypeStruct((b, LANES), jnp.int32),
            jax.ShapeDtypeStruct((b, 16), jnp.float32),
            jax.ShapeDtypeStruct((b, 16), jnp.float32),
            jax.ShapeDtypeStruct((8, LANES), jnp.int32),
        ],
        scratch_shapes=[
            pltpu.VMEM((1, LANES), jnp.float32),
            pltpu.VMEM((1, LANES), jnp.float32),
            pltpu.VMEM((1, LANES), jnp.float32),
        ],
        compiler_params=pltpu.CompilerParams(
            dimension_semantics=("arbitrary", "arbitrary")),
    )(x, wgp, bgp)


def _sc_scatter(x3, pos0, pos1, nrows):
    b, sl, ll = x3.shape
    nw = 32
    chunk = b // nw
    sub = 32
    nsub = chunk // sub
    mesh = plsc.VectorSubcoreMesh(core_axis_name="c", subcore_axis_name="s")

    @functools.partial(
        pl.kernel, mesh=mesh,
        out_type=jax.ShapeDtypeStruct((nrows, sl, ll), jnp.float32),
        scratch_types=[
            pltpu.VMEM((sub, sl, ll), jnp.float32),
            pltpu.VMEM((sub,), jnp.int32),
            pltpu.VMEM((sub,), jnp.int32),
            pltpu.SemaphoreType.DMA,
        ],
    )
    def scat(x_hbm, p0_hbm, p1_hbm, xs_hbm, rows_v, i0_v, i1_v, sem):
        wid = jax.lax.axis_index("s") * 2 + jax.lax.axis_index("c")
        base = wid * chunk

        def body(si, _):
            off = base + si * sub
            pltpu.sync_copy(x_hbm.at[pl.ds(off, sub)], rows_v)
            pltpu.sync_copy(p0_hbm.at[pl.ds(off, sub)], i0_v)
            pltpu.sync_copy(p1_hbm.at[pl.ds(off, sub)], i1_v)
            cp0 = pltpu.async_copy(rows_v, xs_hbm.at[i0_v], sem)
            cp1 = pltpu.async_copy(rows_v, xs_hbm.at[i1_v], sem)
            cp0.wait()
            cp1.wait()
            return ()

        jax.lax.fori_loop(0, nsub, body, ())

    return scat(x3, pos0, pos1)


def _gmm_body(meta_ref, xs_ref, w1_ref, b1_ref, w2_ref, b2_ref, out_ref):
    t = pl.program_id(0)

    @pl.when(meta_ref[1, t] == 1)
    def _compute():
        nk = xs_ref.shape[1]
        acc = jnp.dot(xs_ref[:, 0, :], w1_ref[0, 0],
                      preferred_element_type=jnp.float32)
        for k in range(1, nk):
            acc += jnp.dot(xs_ref[:, k, :], w1_ref[0, k],
                           preferred_element_type=jnp.float32)
        h = jnp.maximum(acc + b1_ref[0], 0.0).astype(jnp.bfloat16)
        out_ref[...] = (jnp.dot(h, w2_ref[0],
                                preferred_element_type=jnp.float32)
                        + b2_ref[0])


def _gmm(meta, xs3, w1r, b1r, w2q, b2q, tmb):
    nrows, nk, ll = xs3.shape
    h = w1r.shape[3]
    ov = w2q.shape[2]
    nt = nrows // tmb
    return pl.pallas_call(
        _gmm_body,
        grid_spec=pltpu.PrefetchScalarGridSpec(
            num_scalar_prefetch=1,
            grid=(nt,),
            in_specs=[
                pl.BlockSpec((tmb, nk, ll), lambda t, meta: (t, 0, 0)),
                pl.BlockSpec((1, nk, ll, h),
                             lambda t, meta: (meta[0, t], 0, 0, 0)),
                pl.BlockSpec((1, 1, h), lambda t, meta: (meta[0, t], 0, 0)),
                pl.BlockSpec((1, h, ov), lambda t, meta: (meta[0, t], 0, 0)),
                pl.BlockSpec((1, 1, ov), lambda t, meta: (meta[0, t], 0, 0)),
            ],
            out_specs=pl.BlockSpec((tmb, ov), lambda t, meta: (t, 0)),
        ),
        out_shape=jax.ShapeDtypeStruct((nrows, ov), jnp.float32),
        compiler_params=pltpu.CompilerParams(
            dimension_semantics=("arbitrary",)),
    )(meta, xs3, w1r, b1r, w2q, b2q)


def _sc_combine(outg, pos0, pos1, wa, wb):
    bk, ov = outg.shape
    b = pos0.shape[0]
    nw = 32
    chunk = b // nw
    mesh = plsc.VectorSubcoreMesh(core_axis_name="c", subcore_axis_name="s")

    @functools.partial(
        pl.kernel, mesh=mesh,
        out_type=jax.ShapeDtypeStruct((b, 16), jnp.float32),
        scratch_types=[
            pltpu.VMEM((chunk, ov), jnp.float32),
            pltpu.VMEM((chunk, ov), jnp.float32),
            pltpu.VMEM((chunk, 16), jnp.float32),
            pltpu.VMEM((chunk,), jnp.int32),
            pltpu.VMEM((chunk,), jnp.int32),
            pltpu.VMEM((chunk, 16), jnp.float32),
            pltpu.VMEM((chunk, 16), jnp.float32),
            pltpu.SemaphoreType.DMA,
        ],
    )
    def comb(outg_hbm, p0_hbm, p1_hbm, wa_hbm, wb_hbm, out_hbm,
             r0_v, r1_v, o_v, i0_v, i1_v, wa_v, wb_v, sem):
        wid = jax.lax.axis_index("s") * 2 + jax.lax.axis_index("c")
        base = wid * chunk
        pltpu.sync_copy(p0_hbm.at[pl.ds(base, chunk)], i0_v)
        pltpu.sync_copy(p1_hbm.at[pl.ds(base, chunk)], i1_v)
        pltpu.sync_copy(wa_hbm.at[pl.ds(base, chunk)], wa_v)
        pltpu.sync_copy(wb_hbm.at[pl.ds(base, chunk)], wb_v)
        cp0 = pltpu.async_copy(outg_hbm.at[i0_v], r0_v, sem)
        cp1 = pltpu.async_copy(outg_hbm.at[i1_v], r1_v, sem)
        cp0.wait()
        cp1.wait()

        def body(i, _):
            o_v[i] = (wa_v[i] * r0_v[i, pl.ds(0, 16)]
                      + wb_v[i] * r1_v[i, pl.ds(0, 16)])
            return ()

        jax.lax.fori_loop(0, chunk, body, ())
        pltpu.sync_copy(o_v, out_hbm.at[pl.ds(base, chunk)])

    return comb(outg, pos0, pos1, wa, wb)


def kernel(x, Wg, bg, W1, b1, W2, b2):
    b, d = x.shape
    n_exp = Wg.shape[1]
    h = W1.shape[2]
    o = W2.shape[2]
    ov = LANES
    tmb = 256
    nrows = 2 * b + n_exp * tmb

    wgp = jnp.pad(Wg, ((0, 0), (0, LANES - n_exp)))
    bgp = jnp.pad(bg, (0, LANES - n_exp), constant_values=-1e30)[None, :]
    w2q = jnp.pad(W2, ((0, 0), (0, 0), (0, ov - o))).astype(jnp.bfloat16)
    b1r = b1[:, None, :]
    b2q = jnp.pad(b2, ((0, 0), (0, ov - o)))[:, None, :]

    gpad, posi, wa, wb, meta = _route(x, wgp, bgp, n_exp, tmb)
    pos0 = posi[:, 0]
    pos1 = posi[:, 1]

    x3 = x.reshape(b, d // LANES, LANES)
    w1r = W1.reshape(n_exp, d // LANES, LANES, h)
    xs3 = _sc_scatter(x3, pos0, pos1, nrows)
    outg = _gmm(meta, xs3, w1r, b1r, w2q, b2q, tmb)
    outw = _sc_combine(outg, pos0, pos1, wa, wb)
    return outw[:, :o], gpad[:, :n_exp]


# speedup vs baseline: 1.2664x; 1.1561x over previous
import functools

import jax
import jax.numpy as jnp
from jax.experimental import pallas as pl
from jax.experimental.pallas import tpu as pltpu
from jax.experimental.pallas import tpu_sc as plsc

LANES = 128


def _route_body(x_ref, wgp_ref, bgp_ref, g_ref, posi_ref, wa_ref, wb_ref,
                meta_ref, cnt_sc, run_sc, off_sc, *, n_exp, tmb):
    p = pl.program_id(0)
    t = pl.program_id(1)
    tm = x_ref.shape[0]
    lanes = jax.lax.broadcasted_iota(jnp.int32, (tm, LANES), 1)

    logits = jnp.dot(x_ref[...], wgp_ref[...],
                     preferred_element_type=jnp.float32) + bgp_ref[...]
    mx = jnp.max(logits, axis=1, keepdims=True)
    ex = jnp.exp(logits - mx)
    gates = ex / jnp.sum(ex, axis=1, keepdims=True)
    m1 = jnp.max(gates, axis=1, keepdims=True)
    i1 = jnp.min(jnp.where(gates == m1, lanes, LANES), axis=1, keepdims=True)
    oh1 = lanes == i1
    rest = jnp.where(oh1, -1.0, gates)
    m2v = jnp.max(rest, axis=1, keepdims=True)
    i2 = jnp.min(jnp.where(rest == m2v, lanes, LANES), axis=1, keepdims=True)
    oh2 = lanes == i2
    oh1f = oh1.astype(jnp.float32)
    oh2f = oh2.astype(jnp.float32)
    ohsum = oh1f + oh2f
    tile_cnt = jnp.sum(ohsum, axis=0, keepdims=True)

    cr = jax.lax.broadcasted_iota(jnp.int32, (LANES, LANES), 0)
    cc = jax.lax.broadcasted_iota(jnp.int32, (LANES, LANES), 1)
    upper = (cr < cc).astype(jnp.float32)

    @pl.when((p == 0) & (t == 0))
    def _init():
        cnt_sc[...] = tile_cnt
        meta_ref[...] = jnp.zeros_like(meta_ref)

    @pl.when((p == 0) & (t > 0))
    def _count():
        cnt_sc[...] += tile_cnt

    @pl.when((p == 1) & (t == 0))
    def _offsets():
        cntp = jnp.ceil(cnt_sc[...] / tmb) * tmb
        off_sc[...] = jnp.dot(cntp, upper,
                              preferred_element_type=jnp.float32,
                              precision=jax.lax.Precision.HIGHEST)
        run_sc[...] = off_sc[...]

    @pl.when(p == 1)
    def _positions():
        s = m1 + m2v + 1e-12
        gm = jnp.where(oh1, m1 / s, 0.0) + jnp.where(oh2, m2v / s, 0.0)
        rr = jax.lax.broadcasted_iota(jnp.int32, (tm, tm), 0)
        rc = jax.lax.broadcasted_iota(jnp.int32, (tm, tm), 1)
        lstrict = (rc < rr).astype(jnp.float32)
        csum = jnp.dot(lstrict, ohsum, preferred_element_type=jnp.float32,
                       precision=jax.lax.Precision.HIGHEST)
        base = run_sc[...] + csum
        pos0 = jnp.sum(base * oh1f, axis=1, keepdims=True).astype(jnp.int32)
        pos1 = jnp.sum(base * oh2f, axis=1, keepdims=True).astype(jnp.int32)
        g_ref[...] = gm
        posi_ref[...] = jnp.where(lanes == 0, pos0,
                                  jnp.where(lanes == 1, pos1, 0))
        wa_ref[...] = jnp.broadcast_to(m1 / s, wa_ref.shape)
        wb_ref[...] = jnp.broadcast_to(m2v / s, wb_ref.shape)
        run_sc[...] += tile_cnt

    @pl.when((p == 1) & (t == pl.num_programs(1) - 1))
    def _meta():
        off = off_sc[...]
        cntp = jnp.ceil(cnt_sc[...] / tmb) * tmb
        ntile_row = cntp / tmb
        tstart_row = off / tmb
        tnext_row = tstart_row + ntile_row
        row8 = jax.lax.broadcasted_iota(jnp.int32, (8, LANES), 0)
        lane8 = jax.lax.broadcasted_iota(jnp.int32, (8, LANES), 1)
        diag = (lane8 == row8).astype(jnp.float32)
        tstart_col = jnp.sum(jnp.broadcast_to(tstart_row, (8, LANES)) * diag,
                             axis=1, keepdims=True)
        tnext_col = jnp.sum(jnp.broadcast_to(tnext_row, (8, LANES)) * diag,
                            axis=1, keepdims=True)
        lanef8 = lane8.astype(jnp.float32)
        belongs = ((lanef8 >= tstart_col) & (lanef8 < tnext_col)
                   & (row8 < n_exp))
        etask = jnp.sum(jnp.where(belongs, row8.astype(jnp.float32), 0.0),
                        axis=0, keepdims=True)
        tot = jnp.sum(ntile_row, axis=1, keepdims=True)
        lane1 = jax.lax.broadcasted_iota(jnp.int32, (1, LANES), 1)
        lanef1 = lane1.astype(jnp.float32)
        elast = jnp.max(jnp.where(cnt_sc[...] > 0, lanef1, 0.0),
                        axis=1, keepdims=True)
        validr = lanef1 < tot
        etaskf = jnp.where(validr, etask, elast)
        metam = jnp.where(
            row8 == 0, jnp.broadcast_to(etaskf, (8, LANES)),
            jnp.where(row8 == 1,
                      jnp.broadcast_to(validr.astype(jnp.float32), (8, LANES)),
                      0.0))
        meta_ref[...] = metam.astype(jnp.int32)


def _route(x, wgp, bgp, n_exp, tmb):
    b, d = x.shape
    tma = min(1024, b)
    body = functools.partial(_route_body, n_exp=n_exp, tmb=tmb)
    return pl.pallas_call(
        body,
        grid=(2, b // tma),
        in_specs=[
            pl.BlockSpec((tma, d), lambda p, t: (t, 0)),
            pl.BlockSpec((d, LANES), lambda p, t: (0, 0)),
            pl.BlockSpec((1, LANES), lambda p, t: (0, 0)),
        ],
        out_specs=[
            pl.BlockSpec((tma, LANES), lambda p, t: (t, 0)),
            pl.BlockSpec((tma, LANES), lambda p, t: (t, 0)),
            pl.BlockSpec((tma, 16), lambda p, t: (t, 0)),
            pl.BlockSpec((tma, 16), lambda p, t: (t, 0)),
            pl.BlockSpec((8, LANES), lambda p, t: (0, 0)),
        ],
        out_shape=[
            jax.ShapeDtypeStruct((b, LANES), jnp.float32),
            jax.ShapeDtypeStruct((b, LANES), jnp.int32),
            jax.ShapeDtypeStruct((b, 16), jnp.float32),
            jax.ShapeDtypeStruct((b, 16), jnp.float32),
            jax.ShapeDtypeStruct((8, LANES), jnp.int32),
        ],
        scratch_shapes=[
            pltpu.VMEM((1, LANES), jnp.float32),
            pltpu.VMEM((1, LANES), jnp.float32),
            pltpu.VMEM((1, LANES), jnp.float32),
        ],
        compiler_params=pltpu.CompilerParams(
            dimension_semantics=("arbitrary", "arbitrary")),
    )(x, wgp, bgp)


def _sc_scatter(x2, pos0, pos1, nrows):
    b, d = x2.shape
    nw = 32
    chunk = b // nw
    sub = 32
    nsub = chunk // sub
    mesh = plsc.VectorSubcoreMesh(core_axis_name="c", subcore_axis_name="s")

    @functools.partial(
        pl.kernel, mesh=mesh,
        out_type=jax.ShapeDtypeStruct((nrows, d), jnp.float32),
        scratch_types=[
            pltpu.VMEM((sub, d), jnp.float32),
            pltpu.VMEM((sub,), jnp.int32),
            pltpu.VMEM((sub,), jnp.int32),
            pltpu.SemaphoreType.DMA,
        ],
    )
    def scat(x_hbm, p0_hbm, p1_hbm, xs_hbm, rows_v, i0_v, i1_v, sem):
        wid = jax.lax.axis_index("s") * 2 + jax.lax.axis_index("c")
        base = wid * chunk

        def body(si, _):
            off = base + si * sub
            pltpu.sync_copy(x_hbm.at[pl.ds(off, sub)], rows_v)
            pltpu.sync_copy(p0_hbm.at[pl.ds(off, sub)], i0_v)
            pltpu.sync_copy(p1_hbm.at[pl.ds(off, sub)], i1_v)
            cp0 = pltpu.async_copy(rows_v, xs_hbm.at[i0_v], sem)
            cp1 = pltpu.async_copy(rows_v, xs_hbm.at[i1_v], sem)
            cp0.wait()
            cp1.wait()
            return ()

        jax.lax.fori_loop(0, nsub, body, ())

    return scat(x2, pos0, pos1)


def _gmm_body(meta_ref, xs_ref, w1_ref, b1_ref, w2_ref, b2_ref, out_ref):
    t = pl.program_id(0)

    @pl.when(meta_ref[1, t] == 1)
    def _compute():
        acc = jnp.dot(xs_ref[...], w1_ref[0],
                      preferred_element_type=jnp.float32)
        h = jnp.maximum(acc + b1_ref[0], 0.0)
        out_ref[...] = (jnp.dot(h, w2_ref[0],
                                preferred_element_type=jnp.float32)
                        + b2_ref[0])


def _gmm(meta, xs2, w1r, b1r, w2q, b2q, tmb):
    nrows, d = xs2.shape
    h = w1r.shape[2]
    ov = w2q.shape[2]
    nt = nrows // tmb
    return pl.pallas_call(
        _gmm_body,
        grid_spec=pltpu.PrefetchScalarGridSpec(
            num_scalar_prefetch=1,
            grid=(nt,),
            in_specs=[
                pl.BlockSpec((tmb, d), lambda t, meta: (t, 0)),
                pl.BlockSpec((1, d, h),
                             lambda t, meta: (meta[0, t], 0, 0)),
                pl.BlockSpec((1, 1, h), lambda t, meta: (meta[0, t], 0, 0)),
                pl.BlockSpec((1, h, ov), lambda t, meta: (meta[0, t], 0, 0)),
                pl.BlockSpec((1, 1, ov), lambda t, meta: (meta[0, t], 0, 0)),
            ],
            out_specs=pl.BlockSpec((tmb, ov), lambda t, meta: (t, 0)),
        ),
        out_shape=jax.ShapeDtypeStruct((nrows, ov), jnp.float32),
        compiler_params=pltpu.CompilerParams(
            dimension_semantics=("arbitrary",)),
    )(meta, xs2, w1r, b1r, w2q, b2q)


def _sc_combine(outg, pos0, pos1, wa, wb):
    bk, ov = outg.shape
    b = pos0.shape[0]
    nw = 32
    chunk = b // nw
    mesh = plsc.VectorSubcoreMesh(core_axis_name="c", subcore_axis_name="s")

    @functools.partial(
        pl.kernel, mesh=mesh,
        out_type=jax.ShapeDtypeStruct((b, 16), jnp.float32),
        scratch_types=[
            pltpu.VMEM((chunk, ov), jnp.float32),
            pltpu.VMEM((chunk, ov), jnp.float32),
            pltpu.VMEM((chunk, 16), jnp.float32),
            pltpu.VMEM((chunk,), jnp.int32),
            pltpu.VMEM((chunk,), jnp.int32),
            pltpu.VMEM((chunk, 16), jnp.float32),
            pltpu.VMEM((chunk, 16), jnp.float32),
            pltpu.SemaphoreType.DMA,
        ],
    )
    def comb(outg_hbm, p0_hbm, p1_hbm, wa_hbm, wb_hbm, out_hbm,
             r0_v, r1_v, o_v, i0_v, i1_v, wa_v, wb_v, sem):
        wid = jax.lax.axis_index("s") * 2 + jax.lax.axis_index("c")
        base = wid * chunk
        pltpu.sync_copy(p0_hbm.at[pl.ds(base, chunk)], i0_v)
        pltpu.sync_copy(p1_hbm.at[pl.ds(base, chunk)], i1_v)
        pltpu.sync_copy(wa_hbm.at[pl.ds(base, chunk)], wa_v)
        pltpu.sync_copy(wb_hbm.at[pl.ds(base, chunk)], wb_v)
        cp0 = pltpu.async_copy(outg_hbm.at[i0_v], r0_v, sem)
        cp1 = pltpu.async_copy(outg_hbm.at[i1_v], r1_v, sem)
        cp0.wait()
        cp1.wait()

        def body(i, _):
            o_v[i] = (wa_v[i] * r0_v[i, pl.ds(0, 16)]
                      + wb_v[i] * r1_v[i, pl.ds(0, 16)])
            return ()

        jax.lax.fori_loop(0, chunk, body, ())
        pltpu.sync_copy(o_v, out_hbm.at[pl.ds(base, chunk)])

    return comb(outg, pos0, pos1, wa, wb)


def kernel(x, Wg, bg, W1, b1, W2, b2):
    b, d = x.shape
    n_exp = Wg.shape[1]
    h = W1.shape[2]
    o = W2.shape[2]
    ov = LANES
    tmb = 256
    nrows = 2 * b + n_exp * tmb

    wgp = jnp.pad(Wg, ((0, 0), (0, LANES - n_exp)))
    bgp = jnp.pad(bg, (0, LANES - n_exp), constant_values=-1e30)[None, :]
    w2q = jnp.pad(W2, ((0, 0), (0, 0), (0, ov - o)))
    b1r = b1[:, None, :]
    b2q = jnp.pad(b2, ((0, 0), (0, ov - o)))[:, None, :]

    gpad, posi, wa, wb, meta = _route(x, wgp, bgp, n_exp, tmb)
    pos0 = posi[:, 0]
    pos1 = posi[:, 1]

    xs2 = _sc_scatter(x, pos0, pos1, nrows)
    outg = _gmm(meta, xs2, W1, b1r, w2q, b2q, tmb)
    outw = _sc_combine(outg, pos0, pos1, wa, wb)
    return outw[:, :o], gpad[:, :n_exp]


# speedup vs baseline: 1.2807x; 1.0113x over previous
import functools

import jax
import jax.numpy as jnp
from jax.experimental import pallas as pl
from jax.experimental.pallas import tpu as pltpu
from jax.experimental.pallas import tpu_sc as plsc

LANES = 128


def _route_body(x_ref, wgp_ref, bgp_ref, g_ref, posi_ref, wa_ref, wb_ref,
                meta_ref, cnt_sc, run_sc, off_sc, gates_sc, *, n_exp, tmb):
    p = pl.program_id(0)
    t = pl.program_id(1)
    tm = x_ref.shape[0]
    lanes = jax.lax.broadcasted_iota(jnp.int32, (tm, LANES), 1)

    def top2(gates):
        m1 = jnp.max(gates, axis=1, keepdims=True)
        i1 = jnp.min(jnp.where(gates == m1, lanes, LANES), axis=1,
                     keepdims=True)
        oh1 = lanes == i1
        rest = jnp.where(oh1, -1.0, gates)
        m2v = jnp.max(rest, axis=1, keepdims=True)
        i2 = jnp.min(jnp.where(rest == m2v, lanes, LANES), axis=1,
                     keepdims=True)
        oh2 = lanes == i2
        oh1f = oh1.astype(jnp.float32)
        oh2f = oh2.astype(jnp.float32)
        ohsum = oh1f + oh2f
        tile_cnt = jnp.sum(ohsum, axis=0, keepdims=True)
        return m1, oh1, oh1f, m2v, oh2, oh2f, ohsum, tile_cnt

    @pl.when(p == 0)
    def _pass0():
        logits = jnp.dot(x_ref[...], wgp_ref[...],
                         preferred_element_type=jnp.float32) + bgp_ref[...]
        mx = jnp.max(logits, axis=1, keepdims=True)
        ex = jnp.exp(logits - mx)
        gates = ex / jnp.sum(ex, axis=1, keepdims=True)
        gates_sc[pl.ds(t * tm, tm)] = gates
        _, _, _, _, _, _, _, tile_cnt = top2(gates)

        @pl.when(t == 0)
        def _init():
            cnt_sc[...] = tile_cnt
            meta_ref[...] = jnp.zeros_like(meta_ref)

        @pl.when(t > 0)
        def _count():
            cnt_sc[...] += tile_cnt

    @pl.when(p == 1)
    def _pass1():
        gates = gates_sc[pl.ds(t * tm, tm)]
        m1, oh1, oh1f, m2v, oh2, oh2f, ohsum, tile_cnt = top2(gates)

        @pl.when(t == 0)
        def _offsets():
            cr = jax.lax.broadcasted_iota(jnp.int32, (LANES, LANES), 0)
            cc = jax.lax.broadcasted_iota(jnp.int32, (LANES, LANES), 1)
            upper = (cr < cc).astype(jnp.float32)
            cntp = jnp.ceil(cnt_sc[...] / tmb) * tmb
            off_sc[...] = jnp.dot(cntp, upper,
                                  preferred_element_type=jnp.float32,
                                  precision=jax.lax.Precision.HIGHEST)
            run_sc[...] = off_sc[...]

        s = m1 + m2v + 1e-12
        gm = jnp.where(oh1, m1 / s, 0.0) + jnp.where(oh2, m2v / s, 0.0)
        rr = jax.lax.broadcasted_iota(jnp.int32, (tm, tm), 0)
        rc = jax.lax.broadcasted_iota(jnp.int32, (tm, tm), 1)
        lstrict = (rc < rr).astype(jnp.float32)
        csum = jnp.dot(lstrict, ohsum, preferred_element_type=jnp.float32,
                       precision=jax.lax.Precision.HIGHEST)
        base = run_sc[...] + csum
        pos0 = jnp.sum(base * oh1f, axis=1, keepdims=True).astype(jnp.int32)
        pos1 = jnp.sum(base * oh2f, axis=1, keepdims=True).astype(jnp.int32)
        g_ref[...] = gm
        posi_ref[...] = jnp.where(lanes == 0, pos0,
                                  jnp.where(lanes == 1, pos1, 0))
        wa_ref[...] = jnp.broadcast_to(m1 / s, wa_ref.shape)
        wb_ref[...] = jnp.broadcast_to(m2v / s, wb_ref.shape)
        run_sc[...] += tile_cnt

    @pl.when((p == 1) & (t == pl.num_programs(1) - 1))
    def _meta():
        off = off_sc[...]
        cntp = jnp.ceil(cnt_sc[...] / tmb) * tmb
        ntile_row = cntp / tmb
        tstart_row = off / tmb
        tnext_row = tstart_row + ntile_row
        row8 = jax.lax.broadcasted_iota(jnp.int32, (8, LANES), 0)
        lane8 = jax.lax.broadcasted_iota(jnp.int32, (8, LANES), 1)
        diag = (lane8 == row8).astype(jnp.float32)
        tstart_col = jnp.sum(jnp.broadcast_to(tstart_row, (8, LANES)) * diag,
                             axis=1, keepdims=True)
        tnext_col = jnp.sum(jnp.broadcast_to(tnext_row, (8, LANES)) * diag,
                            axis=1, keepdims=True)
        lanef8 = lane8.astype(jnp.float32)
        belongs = ((lanef8 >= tstart_col) & (lanef8 < tnext_col)
                   & (row8 < n_exp))
        etask = jnp.sum(jnp.where(belongs, row8.astype(jnp.float32), 0.0),
                        axis=0, keepdims=True)
        tot = jnp.sum(ntile_row, axis=1, keepdims=True)
        lane1 = jax.lax.broadcasted_iota(jnp.int32, (1, LANES), 1)
        lanef1 = lane1.astype(jnp.float32)
        elast = jnp.max(jnp.where(cnt_sc[...] > 0, lanef1, 0.0),
                        axis=1, keepdims=True)
        validr = lanef1 < tot
        etaskf = jnp.where(validr, etask, elast)
        metam = jnp.where(
            row8 == 0, jnp.broadcast_to(etaskf, (8, LANES)),
            jnp.where(row8 == 1,
                      jnp.broadcast_to(validr.astype(jnp.float32), (8, LANES)),
                      0.0))
        meta_ref[...] = metam.astype(jnp.int32)


def _route(x, wgp, bgp, n_exp, tmb):
    b, d = x.shape
    tma = min(1024, b)
    body = functools.partial(_route_body, n_exp=n_exp, tmb=tmb)
    return pl.pallas_call(
        body,
        grid=(2, b // tma),
        in_specs=[
            pl.BlockSpec((tma, d), lambda p, t: ((1 - p) * t, 0)),
            pl.BlockSpec((d, LANES), lambda p, t: (0, 0)),
            pl.BlockSpec((1, LANES), lambda p, t: (0, 0)),
        ],
        out_specs=[
            pl.BlockSpec((tma, LANES), lambda p, t: (t, 0)),
            pl.BlockSpec((tma, LANES), lambda p, t: (t, 0)),
            pl.BlockSpec((tma, 16), lambda p, t: (t, 0)),
            pl.BlockSpec((tma, 16), lambda p, t: (t, 0)),
            pl.BlockSpec((8, LANES), lambda p, t: (0, 0)),
        ],
        out_shape=[
            jax.ShapeDtypeStruct((b, LANES), jnp.float32),
            jax.ShapeDtypeStruct((b, LANES), jnp.int32),
            jax.ShapeDtypeStruct((b, 16), jnp.float32),
            jax.ShapeDtypeStruct((b, 16), jnp.float32),
            jax.ShapeDtypeStruct((8, LANES), jnp.int32),
        ],
        scratch_shapes=[
            pltpu.VMEM((1, LANES), jnp.float32),
            pltpu.VMEM((1, LANES), jnp.float32),
            pltpu.VMEM((1, LANES), jnp.float32),
            pltpu.VMEM((b, LANES), jnp.float32),
        ],
        compiler_params=pltpu.CompilerParams(
            dimension_semantics=("arbitrary", "arbitrary")),
    )(x, wgp, bgp)


def _sc_scatter(x2, pos0, pos1, nrows):
    b, d = x2.shape
    nw = 32
    chunk = b // nw
    sub = 32
    nsub = chunk // sub
    mesh = plsc.VectorSubcoreMesh(core_axis_name="c", subcore_axis_name="s")

    @functools.partial(
        pl.kernel, mesh=mesh,
        out_type=jax.ShapeDtypeStruct((nrows, d), jnp.float32),
        scratch_types=[
            pltpu.VMEM((sub, d), jnp.float32),
            pltpu.VMEM((sub,), jnp.int32),
            pltpu.VMEM((sub,), jnp.int32),
            pltpu.SemaphoreType.DMA,
        ],
    )
    def scat(x_hbm, p0_hbm, p1_hbm, xs_hbm, rows_v, i0_v, i1_v, sem):
        wid = jax.lax.axis_index("s") * 2 + jax.lax.axis_index("c")
        base = wid * chunk

        def body(si, _):
            off = base + si * sub
            pltpu.sync_copy(x_hbm.at[pl.ds(off, sub)], rows_v)
            pltpu.sync_copy(p0_hbm.at[pl.ds(off, sub)], i0_v)
            pltpu.sync_copy(p1_hbm.at[pl.ds(off, sub)], i1_v)
            cp0 = pltpu.async_copy(rows_v, xs_hbm.at[i0_v], sem)
            cp1 = pltpu.async_copy(rows_v, xs_hbm.at[i1_v], sem)
            cp0.wait()
            cp1.wait()
            return ()

        jax.lax.fori_loop(0, nsub, body, ())

    return scat(x2, pos0, pos1)


def _gmm_body(meta_ref, xs_ref, w1_ref, b1_ref, w2_ref, b2_ref, out_ref):
    t = pl.program_id(0)

    @pl.when(meta_ref[1, t] == 1)
    def _compute():
        acc = jnp.dot(xs_ref[...], w1_ref[0],
                      preferred_element_type=jnp.float32)
        h = jnp.maximum(acc + b1_ref[0], 0.0)
        out_ref[...] = (jnp.dot(h, w2_ref[0],
                                preferred_element_type=jnp.float32)
                        + b2_ref[0])


def _gmm(meta, xs2, w1r, b1r, w2q, b2q, tmb):
    nrows, d = xs2.shape
    h = w1r.shape[2]
    ov = w2q.shape[2]
    nt = nrows // tmb
    return pl.pallas_call(
        _gmm_body,
        grid_spec=pltpu.PrefetchScalarGridSpec(
            num_scalar_prefetch=1,
            grid=(nt,),
            in_specs=[
                pl.BlockSpec((tmb, d), lambda t, meta: (t, 0)),
                pl.BlockSpec((1, d, h),
                             lambda t, meta: (meta[0, t], 0, 0)),
                pl.BlockSpec((1, 1, h), lambda t, meta: (meta[0, t], 0, 0)),
                pl.BlockSpec((1, h, ov), lambda t, meta: (meta[0, t], 0, 0)),
                pl.BlockSpec((1, 1, ov), lambda t, meta: (meta[0, t], 0, 0)),
            ],
            out_specs=pl.BlockSpec((tmb, ov), lambda t, meta: (t, 0)),
        ),
        out_shape=jax.ShapeDtypeStruct((nrows, ov), jnp.float32),
        compiler_params=pltpu.CompilerParams(
            dimension_semantics=("arbitrary",)),
    )(meta, xs2, w1r, b1r, w2q, b2q)


def _sc_combine(outg, pos0, pos1, wa, wb):
    bk, ov = outg.shape
    b = pos0.shape[0]
    nw = 32
    chunk = b // nw
    mesh = plsc.VectorSubcoreMesh(core_axis_name="c", subcore_axis_name="s")

    @functools.partial(
        pl.kernel, mesh=mesh,
        out_type=jax.ShapeDtypeStruct((b, 16), jnp.float32),
        scratch_types=[
            pltpu.VMEM((chunk, ov), jnp.float32),
            pltpu.VMEM((chunk, ov), jnp.float32),
            pltpu.VMEM((chunk, 16), jnp.float32),
            pltpu.VMEM((chunk,), jnp.int32),
            pltpu.VMEM((chunk,), jnp.int32),
            pltpu.VMEM((chunk, 16), jnp.float32),
            pltpu.VMEM((chunk, 16), jnp.float32),
            pltpu.SemaphoreType.DMA,
        ],
    )
    def comb(outg_hbm, p0_hbm, p1_hbm, wa_hbm, wb_hbm, out_hbm,
             r0_v, r1_v, o_v, i0_v, i1_v, wa_v, wb_v, sem):
        wid = jax.lax.axis_index("s") * 2 + jax.lax.axis_index("c")
        base = wid * chunk
        pltpu.sync_copy(p0_hbm.at[pl.ds(base, chunk)], i0_v)
        pltpu.sync_copy(p1_hbm.at[pl.ds(base, chunk)], i1_v)
        pltpu.sync_copy(wa_hbm.at[pl.ds(base, chunk)], wa_v)
        pltpu.sync_copy(wb_hbm.at[pl.ds(base, chunk)], wb_v)
        cp0 = pltpu.async_copy(outg_hbm.at[i0_v], r0_v, sem)
        cp1 = pltpu.async_copy(outg_hbm.at[i1_v], r1_v, sem)
        cp0.wait()
        cp1.wait()

        o_v[...] = (wa_v[...] * r0_v[:, pl.ds(0, 16)]
                    + wb_v[...] * r1_v[:, pl.ds(0, 16)])
        pltpu.sync_copy(o_v, out_hbm.at[pl.ds(base, chunk)])

    return comb(outg, pos0, pos1, wa, wb)


def kernel(x, Wg, bg, W1, b1, W2, b2):
    b, d = x.shape
    n_exp = Wg.shape[1]
    h = W1.shape[2]
    o = W2.shape[2]
    ov = LANES
    tmb = 256
    nrows = 2 * b + n_exp * tmb

    wgp = jnp.pad(Wg, ((0, 0), (0, LANES - n_exp)))
    bgp = jnp.pad(bg, (0, LANES - n_exp), constant_values=-1e30)[None, :]
    w2q = jnp.pad(W2, ((0, 0), (0, 0), (0, ov - o)))
    b1r = b1[:, None, :]
    b2q = jnp.pad(b2, ((0, 0), (0, ov - o)))[:, None, :]

    gpad, posi, wa, wb, meta = _route(x, wgp, bgp, n_exp, tmb)
    pos0 = posi[:, 0]
    pos1 = posi[:, 1]

    xs2 = _sc_scatter(x, pos0, pos1, nrows)
    outg = _gmm(meta, xs2, W1, b1r, w2q, b2q, tmb)
    outw = _sc_combine(outg, pos0, pos1, wa, wb)
    return outw[:, :o], gpad[:, :n_exp]
